# baseline (speedup 1.0000x reference)
import jax
import jax.numpy as jnp
from jax import lax
from jax.experimental import pallas as pl

B, S, DEPTH = 1024, 50, 1000
DB = 200


def _onehot_t_body(idx_ref, out_ref):
    j = pl.program_id(1)
    row = idx_ref[0, 0, :]
    d_iota = j * DB + lax.broadcasted_iota(jnp.int32, (DB, B), 0)
    out_ref[0] = (row[None, :] == d_iota).astype(jnp.float32)


def kernel(inputs):
    idx_t = inputs.astype(jnp.int32).T.reshape(S, 1, B)
    out_t = pl.pallas_call(
        _onehot_t_body,
        grid=(S, DEPTH // DB),
        in_specs=[pl.BlockSpec((1, 1, B), lambda i, j: (i, 0, 0))],
        out_specs=pl.BlockSpec((1, DB, B), lambda i, j: (i, j, 0)),
        out_shape=jax.ShapeDtypeStruct((S, DEPTH, B), jnp.float32),
    )(idx_t)
    return out_t.transpose(2, 0, 1)


# speedup vs baseline: 2.1269x; 2.1269x over previous
import jax
import jax.numpy as jnp
from jax import lax
from jax.experimental import pallas as pl

B, S, DEPTH = 1024, 50, 1000


def _onehot_t_body(idx_ref, out_ref):
    row = idx_ref[0, 0, :]
    d_iota = lax.broadcasted_iota(jnp.int32, (DEPTH, B), 0)
    out_ref[0] = (row[None, :] == d_iota).astype(jnp.float32)


def kernel(inputs):
    idx_t = inputs.astype(jnp.int32).T.reshape(S, 1, B)
    out_t = pl.pallas_call(
        _onehot_t_body,
        grid=(S,),
        in_specs=[pl.BlockSpec((1, 1, B), lambda i: (i, 0, 0))],
        out_specs=pl.BlockSpec((1, DEPTH, B), lambda i: (i, 0, 0)),
        out_shape=jax.ShapeDtypeStruct((S, DEPTH, B), jnp.float32),
    )(idx_t)
    return out_t.transpose(2, 0, 1)
